# Initial kernel scaffold; baseline (speedup 1.0000x reference)
import jax
import jax.numpy as jnp
from jax.experimental import pallas as pl


def kernel(x, edge_index, batch, emb_table, bn_gamma, bn_beta, W, b, g1, be1, W1, bb1, g2, be2, W2, bb2):
    raise NotImplementedError("write your pallas kernel here")


# speedup vs baseline: 10.6503x; 10.6503x over previous
import jax
import jax.numpy as jnp
import numpy as np
from jax import lax
from jax.experimental import pallas as pl
from jax.experimental.pallas import tpu as pltpu
from jax.experimental.pallas import tpu_sc as plsc

_ATOM_DIMS = [119, 5, 12, 12, 10, 6, 6, 2, 2]
_SIZES = [d + 1 for d in _ATOM_DIMS]
_OFFS = np.concatenate([[0], np.cumsum(_SIZES)[:-1]]).astype(np.int32)
_TOTAL_ROWS = int(sum(_SIZES))
_N = 10000
_E = 320000
_H = 128
_G = 256
_C = 10
_EPS = 1e-5

_NC = 2
_NS = 16
_NW = _NC * _NS
_EPW = _E // _NW
_K = 80
_NCHUNK = _EPW // _K
_NP = 10240
_RPT = _NP // _NS
_NB = _RPT // _K


def _mesh():
    return plsc.VectorSubcoreMesh(
        core_axis_name="c", subcore_axis_name="s", num_cores=_NC,
        num_subcores=_NS)


def _agg_body(src_ref, dst_ref, u_ref, zeros_ref, out_ref,
              idx_s, idx_d, rows, sem, shared):
    c = lax.axis_index("c")
    s = lax.axis_index("s")
    wid = c * _NS + s
    base = wid * _EPW
    row0 = s * _RPT

    pltpu.sync_copy(zeros_ref, rows)
    for t in range(_NB):
        pltpu.sync_copy(rows, shared.at[pl.ds(row0 + t * _K, _K), :])
    plsc.subcore_barrier()

    def chunk(j, _):
        off = base + j * _K
        pltpu.sync_copy(src_ref.at[pl.ds(off, _K)], idx_s)
        pltpu.sync_copy(dst_ref.at[pl.ds(off, _K)], idx_d)
        pltpu.async_copy(u_ref.at[idx_s], rows, sem).wait()
        pltpu.sync_copy(rows, shared.at[idx_d], add=True)
        return 0

    lax.fori_loop(0, _NCHUNK, chunk, 0)
    plsc.subcore_barrier()
    for t in range(_NB):
        pltpu.sync_copy(shared.at[pl.ds(row0 + t * _K, _K), :], rows)
        pltpu.sync_copy(rows, out_ref.at[c, pl.ds(row0 + t * _K, _K), :])


def _sc_edge_agg(src, dst, u, zeros):
    return pl.kernel(
        _agg_body,
        out_type=jax.ShapeDtypeStruct((_NC, _NP, _H), jnp.float32),
        mesh=_mesh(),
        scratch_types=[
            pltpu.VMEM((_K,), jnp.int32),
            pltpu.VMEM((_K,), jnp.int32),
            pltpu.VMEM((_K, _H), jnp.float32),
            pltpu.SemaphoreType.DMA,
            pltpu.VMEM_SHARED((_NP, _H), jnp.float32),
        ],
    )(src, dst, u, zeros)


def _deg_body(dst_ref, onesrow_ref, zeros_ref, out_ref, idx_d, rows, shared):
    c = lax.axis_index("c")
    s = lax.axis_index("s")
    wid = c * _NS + s
    base = wid * _EPW
    row0 = s * _RPT

    pltpu.sync_copy(zeros_ref, rows)
    for t in range(_NB):
        pltpu.sync_copy(rows, shared.at[pl.ds(row0 + t * _K, _K), :])
    pltpu.sync_copy(onesrow_ref, rows)
    plsc.subcore_barrier()

    def chunk(j, _):
        pltpu.sync_copy(dst_ref.at[pl.ds(base + j * _K, _K)], idx_d)
        pltpu.sync_copy(rows, shared.at[idx_d], add=True)
        return 0

    lax.fori_loop(0, _NCHUNK, chunk, 0)
    plsc.subcore_barrier()
    for t in range(_NB):
        pltpu.sync_copy(shared.at[pl.ds(row0 + t * _K, _K), :], rows)
        pltpu.sync_copy(rows, out_ref.at[c, pl.ds(row0 + t * _K, _K), :])


def _sc_degree(dst, onesrow, zeros16):
    return pl.kernel(
        _deg_body,
        out_type=jax.ShapeDtypeStruct((_NC, _NP, 16), jnp.float32),
        mesh=_mesh(),
        scratch_types=[
            pltpu.VMEM((_K,), jnp.int32),
            pltpu.VMEM((_K, 16), jnp.float32),
            pltpu.VMEM_SHARED((_NP, 16), jnp.float32),
        ],
    )(dst, onesrow, zeros16)


def _embed_body(x_ref, emb_ref, out_ref):
    xv = x_ref[...]
    col = lax.broadcasted_iota(jnp.int32, (_N, 256), 1)
    cnt = jnp.zeros((_N, 256), jnp.float32)
    for j in range(9):
        cnt = cnt + (xv[:, j:j + 1] + int(_OFFS[j]) == col).astype(jnp.float32)
    out_ref[...] = jnp.dot(cnt, emb_ref[...],
                           preferred_element_type=jnp.float32)


def _tc_embed(x, emb_pad):
    return pl.pallas_call(
        _embed_body,
        out_shape=jax.ShapeDtypeStruct((_N, _H), jnp.float32),
    )(x, emb_pad)


def _bn(h, gamma, beta):
    mu = jnp.mean(h, axis=0, keepdims=True)
    var = jnp.mean((h - mu) * (h - mu), axis=0, keepdims=True)
    return (h - mu) * lax.rsqrt(var + _EPS) * gamma + beta


def _layer0_body(h_ref, degp_ref, gamma_ref, beta_ref, w_ref,
                 u_ref, dinv_ref):
    deg = degp_ref[0] + degp_ref[1] + 1.0
    dinv = lax.rsqrt(deg)
    dinv_ref[...] = dinv
    hn = _bn(h_ref[...], gamma_ref[...], beta_ref[...])
    u_ref[...] = jnp.dot(hn, w_ref[...],
                         preferred_element_type=jnp.float32) * dinv


def _tc_layer0(h0, degp, gamma, beta, w):
    return pl.pallas_call(
        _layer0_body,
        out_shape=[
            jax.ShapeDtypeStruct((_N, _H), jnp.float32),
            jax.ShapeDtypeStruct((_N, 1), jnp.float32),
        ],
    )(h0, degp, gamma, beta, w)


def _layer_body(p_ref, u_ref, dinv_ref, bprev_ref, gamma_ref, beta_ref,
                w_ref, out_ref):
    dinv = dinv_ref[...]
    h = jnp.maximum(
        (p_ref[0] + p_ref[1] + u_ref[...]) * dinv + bprev_ref[...], 0.0)
    hn = _bn(h, gamma_ref[...], beta_ref[...])
    out_ref[...] = jnp.dot(hn, w_ref[...],
                           preferred_element_type=jnp.float32) * dinv


def _tc_layer(p, u, dinv, bprev, gamma, beta, w):
    return pl.pallas_call(
        _layer_body,
        out_shape=jax.ShapeDtypeStruct((_N, _H), jnp.float32),
    )(p, u, dinv, bprev, gamma, beta, w)


def _readout_body(p_ref, u_ref, dinv_ref, b2_ref, batch_ref,
                  g1_ref, be1_ref, w1_ref, bb1_ref,
                  g2_ref, be2_ref, w2_ref, bb2_ref, out_ref):
    dinv = dinv_ref[...]
    h3 = jnp.maximum(
        (p_ref[0] + p_ref[1] + u_ref[...]) * dinv + b2_ref[...], 0.0)
    gid = lax.broadcasted_iota(jnp.int32, (_G, _N), 0)
    oh = (gid == batch_ref[...]).astype(jnp.float32)
    pooled = jnp.dot(oh, h3, preferred_element_type=jnp.float32)
    z = _bn(pooled, g1_ref[...], be1_ref[...])
    z = jnp.maximum(jnp.dot(z, w1_ref[...],
                            preferred_element_type=jnp.float32)
                    + bb1_ref[...], 0.0)
    z = _bn(z, g2_ref[...], be2_ref[...])
    out_ref[...] = jnp.dot(z, w2_ref[...],
                           preferred_element_type=jnp.float32) + bb2_ref[...]


def _tc_readout(p, u, dinv, b2, batch2, g1, be1, w1, bb1, g2, be2, w2, bb2):
    return pl.pallas_call(
        _readout_body,
        out_shape=jax.ShapeDtypeStruct((_G, _C), jnp.float32),
    )(p, u, dinv, b2, batch2, g1, be1, w1, bb1, g2, be2, w2, bb2)


def kernel(x, edge_index, batch, emb_table, bn_gamma, bn_beta, W, b,
           g1, be1, W1, bb1, g2, be2, W2, bb2):
    x = x.astype(jnp.int32)
    edge_index = edge_index.astype(jnp.int32)
    src = edge_index[0]
    dst = edge_index[1]
    batch2 = batch.astype(jnp.int32).reshape(1, _N)
    emb_pad = jnp.zeros((256, _H), jnp.float32).at[:_TOTAL_ROWS].set(emb_table)
    zeros = jnp.zeros((_K, _H), jnp.float32)
    zeros16 = jnp.zeros((_K, 16), jnp.float32)
    onesrow = jnp.zeros((_K, 16), jnp.float32).at[:, 0].set(1.0)

    h0 = _tc_embed(x, emb_pad)
    degw = _sc_degree(dst, onesrow, zeros16)
    degp = degw[:, :_N, 0:1]

    u0, dinv = _tc_layer0(h0, degp, bn_gamma[0], bn_beta[0], W[0])
    p0 = _sc_edge_agg(src, dst, u0, zeros)[:, :_N]
    u1 = _tc_layer(p0, u0, dinv, b[0], bn_gamma[1], bn_beta[1], W[1])
    p1 = _sc_edge_agg(src, dst, u1, zeros)[:, :_N]
    u2 = _tc_layer(p1, u1, dinv, b[1], bn_gamma[2], bn_beta[2], W[2])
    p2 = _sc_edge_agg(src, dst, u2, zeros)[:, :_N]
    return _tc_readout(p2, u2, dinv, b[2], batch2,
                       g1, be1, W1, bb1, g2, be2, W2, bb2)


# speedup vs baseline: 15.2935x; 1.4360x over previous
import jax
import jax.numpy as jnp
import numpy as np
from jax import lax
from jax.experimental import pallas as pl
from jax.experimental.pallas import tpu as pltpu
from jax.experimental.pallas import tpu_sc as plsc

_ATOM_DIMS = [119, 5, 12, 12, 10, 6, 6, 2, 2]
_SIZES = [d + 1 for d in _ATOM_DIMS]
_OFFS = np.concatenate([[0], np.cumsum(_SIZES)[:-1]]).astype(np.int32)
_TOTAL_ROWS = int(sum(_SIZES))
_N = 10000
_E = 320000
_H = 128
_G = 256
_C = 10
_EPS = 1e-5

_NC = 2
_NS = 16
_NW = _NC * _NS
_EPW = _E // _NW
_K = 80
_NCHUNK = _EPW // _K
_NP = 10240
_RPT = _NP // _NS
_NBF = _RPT // _K
_NBR = _RPT - _NBF * _K


def _mesh():
    return plsc.VectorSubcoreMesh(
        core_axis_name="c", subcore_axis_name="s", num_cores=_NC,
        num_subcores=_NS)


def _zero_slice(zeros_ref, rows, shared, row0):
    pltpu.sync_copy(zeros_ref, rows)
    for t in range(_NBF):
        pltpu.sync_copy(rows, shared.at[pl.ds(row0 + t * _K, _K), :])
    if _NBR:
        pltpu.sync_copy(rows.at[pl.ds(0, _NBR), :],
                        shared.at[pl.ds(row0 + _NBF * _K, _NBR), :])


def _write_slice(shared, out_ref, rows, c, row0):
    for t in range(_NBF):
        pltpu.sync_copy(shared.at[pl.ds(row0 + t * _K, _K), :], rows)
        pltpu.sync_copy(rows, out_ref.at[c, pl.ds(row0 + t * _K, _K), :])
    if _NBR:
        pltpu.sync_copy(shared.at[pl.ds(row0 + _NBF * _K, _NBR), :],
                        rows.at[pl.ds(0, _NBR), :])
        pltpu.sync_copy(rows.at[pl.ds(0, _NBR), :],
                        out_ref.at[c, pl.ds(row0 + _NBF * _K, _NBR), :])


def _agg_body(src_ref, dst_ref, u_ref, zeros_ref, out_ref,
              ixs_a, ixd_a, ixs_b, ixd_b, rows_a, rows_b,
              semi_a, semi_b, semg_a, semg_b, shared):
    c = lax.axis_index("c")
    s = lax.axis_index("s")
    wid = c * _NS + s
    row0 = s * _RPT
    base = wid * _EPW

    _zero_slice(zeros_ref, rows_a, shared, row0)
    plsc.subcore_barrier()

    def start_idx(j, ixs, ixd, semi):
        off = base + j * _K
        pltpu.make_async_copy(src_ref.at[pl.ds(off, _K)], ixs, semi).start()
        pltpu.make_async_copy(dst_ref.at[pl.ds(off, _K)], ixd, semi).start()

    def wait_idx(j, ixs, ixd, semi):
        off = base + j * _K
        pltpu.make_async_copy(src_ref.at[pl.ds(off, _K)], ixs, semi).wait()
        pltpu.make_async_copy(dst_ref.at[pl.ds(off, _K)], ixd, semi).wait()

    def scatter(rows, ixd):
        pltpu.sync_copy(rows, shared.at[ixd], add=True)

    start_idx(0, ixs_a, ixd_a, semi_a)
    start_idx(1, ixs_b, ixd_b, semi_b)

    def gather_scatter(rows, ixs, ixd, semg):
        g = pltpu.make_async_copy(u_ref.at[ixs], rows, semg)
        g.start()
        g.wait()
        scatter(rows, ixd)

    def pair(jj, _):
        j0 = jj * 2
        j1 = j0 + 1
        wait_idx(j0, ixs_a, ixd_a, semi_a)
        gather_scatter(rows_a, ixs_a, ixd_a, semg_a)
        start_idx(j0 + 2, ixs_a, ixd_a, semi_a)
        wait_idx(j1, ixs_b, ixd_b, semi_b)
        gather_scatter(rows_b, ixs_b, ixd_b, semg_b)
        start_idx(j1 + 2, ixs_b, ixd_b, semi_b)
        return 0

    lax.fori_loop(0, (_NCHUNK - 3) // 2, pair, 0)
    wait_idx(_NCHUNK - 3, ixs_a, ixd_a, semi_a)
    gather_scatter(rows_a, ixs_a, ixd_a, semg_a)
    start_idx(_NCHUNK - 1, ixs_a, ixd_a, semi_a)
    wait_idx(_NCHUNK - 2, ixs_b, ixd_b, semi_b)
    gather_scatter(rows_b, ixs_b, ixd_b, semg_b)
    wait_idx(_NCHUNK - 1, ixs_a, ixd_a, semi_a)
    gather_scatter(rows_a, ixs_a, ixd_a, semg_a)

    plsc.subcore_barrier()
    _write_slice(shared, out_ref, rows_a, c, row0)


def _sc_edge_agg(src, dst, u, zeros):
    return pl.kernel(
        _agg_body,
        out_type=jax.ShapeDtypeStruct((_NC, _NP, _H), jnp.float32),
        mesh=_mesh(),
        scratch_types=[
            pltpu.VMEM((_K,), jnp.int32),
            pltpu.VMEM((_K,), jnp.int32),
            pltpu.VMEM((_K,), jnp.int32),
            pltpu.VMEM((_K,), jnp.int32),
            pltpu.VMEM((_K, _H), jnp.float32),
            pltpu.VMEM((_K, _H), jnp.float32),
            pltpu.SemaphoreType.DMA,
            pltpu.SemaphoreType.DMA,
            pltpu.SemaphoreType.DMA,
            pltpu.SemaphoreType.DMA,
            pltpu.VMEM_SHARED((_NP, _H), jnp.float32),
        ],
    )(src, dst, u, zeros)


def _deg_body(dst_ref, onesrow_ref, zeros_ref, out_ref,
              ixd_a, ixd_b, rows, semi_a, semi_b, shared):
    c = lax.axis_index("c")
    s = lax.axis_index("s")
    wid = c * _NS + s
    row0 = s * _RPT
    base = wid * _EPW

    _zero_slice(zeros_ref, rows, shared, row0)
    pltpu.sync_copy(onesrow_ref, rows)
    plsc.subcore_barrier()

    def start_idx(j, ixd, semi):
        pltpu.make_async_copy(dst_ref.at[pl.ds(base + j * _K, _K)],
                              ixd, semi).start()

    def wait_idx(j, ixd, semi):
        pltpu.make_async_copy(dst_ref.at[pl.ds(base + j * _K, _K)],
                              ixd, semi).wait()

    start_idx(0, ixd_a, semi_a)
    start_idx(1, ixd_b, semi_b)

    def pair(jj, _):
        j0 = jj * 2
        j1 = j0 + 1
        wait_idx(j0, ixd_a, semi_a)
        pltpu.sync_copy(rows, shared.at[ixd_a], add=True)
        start_idx(j0 + 2, ixd_a, semi_a)
        wait_idx(j1, ixd_b, semi_b)
        pltpu.sync_copy(rows, shared.at[ixd_b], add=True)
        start_idx(j1 + 2, ixd_b, semi_b)
        return 0

    lax.fori_loop(0, (_NCHUNK - 3) // 2, pair, 0)
    wait_idx(_NCHUNK - 3, ixd_a, semi_a)
    pltpu.sync_copy(rows, shared.at[ixd_a], add=True)
    start_idx(_NCHUNK - 1, ixd_a, semi_a)
    wait_idx(_NCHUNK - 2, ixd_b, semi_b)
    pltpu.sync_copy(rows, shared.at[ixd_b], add=True)
    wait_idx(_NCHUNK - 1, ixd_a, semi_a)
    pltpu.sync_copy(rows, shared.at[ixd_a], add=True)

    plsc.subcore_barrier()
    _write_slice(shared, out_ref, rows, c, row0)


def _sc_degree(dst, onesrow, zeros16):
    return pl.kernel(
        _deg_body,
        out_type=jax.ShapeDtypeStruct((_NC, _NP, 16), jnp.float32),
        mesh=_mesh(),
        scratch_types=[
            pltpu.VMEM((_K,), jnp.int32),
            pltpu.VMEM((_K,), jnp.int32),
            pltpu.VMEM((_K, 16), jnp.float32),
            pltpu.SemaphoreType.DMA,
            pltpu.SemaphoreType.DMA,
            pltpu.VMEM_SHARED((_NP, 16), jnp.float32),
        ],
    )(dst, onesrow, zeros16)


def _embed_body(x_ref, emb_ref, out_ref):
    xv = x_ref[...]
    col = lax.broadcasted_iota(jnp.int32, (_N, 256), 1)
    cnt = jnp.zeros((_N, 256), jnp.float32)
    for j in range(9):
        cnt = cnt + (xv[:, j:j + 1] + int(_OFFS[j]) == col).astype(jnp.float32)
    out_ref[...] = jnp.dot(cnt, emb_ref[...],
                           preferred_element_type=jnp.float32)


def _tc_embed(x, emb_pad):
    return pl.pallas_call(
        _embed_body,
        out_shape=jax.ShapeDtypeStruct((_N, _H), jnp.float32),
    )(x, emb_pad)


def _bn(h, gamma, beta):
    mu = jnp.mean(h, axis=0, keepdims=True)
    var = jnp.mean((h - mu) * (h - mu), axis=0, keepdims=True)
    return (h - mu) * lax.rsqrt(var + _EPS) * gamma + beta


def _layer0_body(h_ref, degp_ref, gamma_ref, beta_ref, w_ref,
                 u_ref, dinv_ref):
    deg = degp_ref[0] + degp_ref[1] + 1.0
    dinv = lax.rsqrt(deg)
    dinv_ref[...] = dinv
    hn = _bn(h_ref[...], gamma_ref[...], beta_ref[...])
    u_ref[...] = jnp.dot(hn, w_ref[...],
                         preferred_element_type=jnp.float32) * dinv


def _tc_layer0(h0, degp, gamma, beta, w):
    return pl.pallas_call(
        _layer0_body,
        out_shape=[
            jax.ShapeDtypeStruct((_N, _H), jnp.float32),
            jax.ShapeDtypeStruct((_N, 1), jnp.float32),
        ],
    )(h0, degp, gamma, beta, w)


def _layer_body(p_ref, u_ref, dinv_ref, bprev_ref, gamma_ref, beta_ref,
                w_ref, out_ref):
    dinv = dinv_ref[...]
    h = jnp.maximum(
        (p_ref[0] + p_ref[1] + u_ref[...]) * dinv + bprev_ref[...], 0.0)
    hn = _bn(h, gamma_ref[...], beta_ref[...])
    out_ref[...] = jnp.dot(hn, w_ref[...],
                           preferred_element_type=jnp.float32) * dinv


def _tc_layer(p, u, dinv, bprev, gamma, beta, w):
    return pl.pallas_call(
        _layer_body,
        out_shape=jax.ShapeDtypeStruct((_N, _H), jnp.float32),
    )(p, u, dinv, bprev, gamma, beta, w)


def _readout_body(p_ref, u_ref, dinv_ref, b2_ref, batch_ref,
                  g1_ref, be1_ref, w1_ref, bb1_ref,
                  g2_ref, be2_ref, w2_ref, bb2_ref, out_ref):
    dinv = dinv_ref[...]
    h3 = jnp.maximum(
        (p_ref[0] + p_ref[1] + u_ref[...]) * dinv + b2_ref[...], 0.0)
    gid = lax.broadcasted_iota(jnp.int32, (_G, _N), 0)
    oh = (gid == batch_ref[...]).astype(jnp.float32)
    pooled = jnp.dot(oh, h3, preferred_element_type=jnp.float32)
    z = _bn(pooled, g1_ref[...], be1_ref[...])
    z = jnp.maximum(jnp.dot(z, w1_ref[...],
                            preferred_element_type=jnp.float32)
                    + bb1_ref[...], 0.0)
    z = _bn(z, g2_ref[...], be2_ref[...])
    out_ref[...] = jnp.dot(z, w2_ref[...],
                           preferred_element_type=jnp.float32) + bb2_ref[...]


def _tc_readout(p, u, dinv, b2, batch2, g1, be1, w1, bb1, g2, be2, w2, bb2):
    return pl.pallas_call(
        _readout_body,
        out_shape=jax.ShapeDtypeStruct((_G, _C), jnp.float32),
    )(p, u, dinv, b2, batch2, g1, be1, w1, bb1, g2, be2, w2, bb2)


def kernel(x, edge_index, batch, emb_table, bn_gamma, bn_beta, W, b,
           g1, be1, W1, bb1, g2, be2, W2, bb2):
    x = x.astype(jnp.int32)
    edge_index = edge_index.astype(jnp.int32)
    src = edge_index[0]
    dst = edge_index[1]
    batch2 = batch.astype(jnp.int32).reshape(1, _N)
    emb_pad = jnp.zeros((256, _H), jnp.float32).at[:_TOTAL_ROWS].set(emb_table)
    zeros = jnp.zeros((_K, _H), jnp.float32)
    zeros16 = jnp.zeros((_K, 16), jnp.float32)
    onesrow = jnp.zeros((_K, 16), jnp.float32).at[:, 0].set(1.0)

    h0 = _tc_embed(x, emb_pad)
    degw = _sc_degree(dst, onesrow, zeros16)
    degp = degw[:, :_N, 0:1]

    u0, dinv = _tc_layer0(h0, degp, bn_gamma[0], bn_beta[0], W[0])
    p0 = _sc_edge_agg(src, dst, u0, zeros)[:, :_N]
    u1 = _tc_layer(p0, u0, dinv, b[0], bn_gamma[1], bn_beta[1], W[1])
    p1 = _sc_edge_agg(src, dst, u1, zeros)[:, :_N]
    u2 = _tc_layer(p1, u1, dinv, b[1], bn_gamma[2], bn_beta[2], W[2])
    p2 = _sc_edge_agg(src, dst, u2, zeros)[:, :_N]
    return _tc_readout(p2, u2, dinv, b[2], batch2,
                       g1, be1, W1, bb1, g2, be2, W2, bb2)
